# Initial kernel scaffold; baseline (speedup 1.0000x reference)
import jax
import jax.numpy as jnp
from jax.experimental import pallas as pl


def kernel(x, edge_index, edge_weight, h0, W_z, b_z, W_r, b_r, W_h, b_h, lz_w, lz_b, lr_w, lr_b, lh_w, lh_b, lin_w, lin_b):
    raise NotImplementedError("write your pallas kernel here")


# speedup vs baseline: 11.9600x; 11.9600x over previous
import functools

import jax
import jax.numpy as jnp
from jax import lax
from jax.experimental import pallas as pl
from jax.experimental.pallas import tpu as pltpu
from jax.experimental.pallas import tpu_sc as plsc

N = 10000
E = 320000
D = 128

NC = 2
NS = 16
EPT = E // (NC * NS)
CH = 80
NCHUNK = EPT // CH
NP = 10240
RPT = NP // NS

_mesh = plsc.VectorSubcoreMesh(core_axis_name="c", subcore_axis_name="s")


def _wpad_body(w_ref, w16_ref):
    w16_ref[...] = jnp.broadcast_to(w_ref[...], w16_ref.shape)


@functools.partial(
    pl.kernel,
    mesh=_mesh,
    out_type=jax.ShapeDtypeStruct((NC, NP, 16), jnp.float32),
    scratch_types=[
        pltpu.VMEM_SHARED((NP, 16), jnp.float32),
        pltpu.VMEM((1, CH), jnp.int32),
        pltpu.VMEM((CH, 16), jnp.float32),
        pltpu.VMEM((RPT, 16), jnp.float32),
    ],
)
def _deg_call(dst_hbm, w16_hbm, out_hbm, deg_sh, dst2d, wchunk, zbuf):
    c = lax.axis_index("c")
    t = lax.axis_index("s")

    zero16 = jnp.zeros((16,), jnp.float32)

    @pl.loop(0, RPT)
    def _(r):
        zbuf[r, :] = zero16

    pltpu.sync_copy(zbuf, deg_sh.at[pl.ds(t * RPT, RPT)])
    plsc.subcore_barrier()

    base = (c * NS + t) * EPT

    @pl.loop(0, NCHUNK)
    def _(i):
        off = base + i * CH
        pltpu.sync_copy(dst_hbm.at[pl.ds(off, CH)], dst2d.at[0])
        pltpu.sync_copy(w16_hbm.at[pl.ds(off, CH)], wchunk)
        pltpu.sync_copy(wchunk, deg_sh.at[dst2d.at[0]], add=True)

    plsc.subcore_barrier()
    pltpu.sync_copy(deg_sh.at[pl.ds(t * RPT, RPT)], zbuf)
    pltpu.sync_copy(zbuf, out_hbm.at[c, pl.ds(t * RPT, RPT)])


@functools.partial(
    pl.kernel,
    mesh=_mesh,
    out_type=jax.ShapeDtypeStruct((NC, NP, D), jnp.float32),
    scratch_types=[
        pltpu.VMEM_SHARED((NP, D), jnp.float32),
        pltpu.VMEM((2, CH), jnp.int32),
        pltpu.VMEM((CH, 16), jnp.float32),
        pltpu.VMEM((CH, D), jnp.float32),
        pltpu.VMEM((RPT // 5, D), jnp.float32),
    ],
)
def _prop_call(xt_hbm, src_hbm, dst_hbm, w16_hbm, out_hbm,
               acc_sh, idx2d, wchunk, rows, zbuf):
    c = lax.axis_index("c")
    t = lax.axis_index("s")
    zp = RPT // 5

    zero16 = jnp.zeros((16,), jnp.float32)

    @pl.loop(0, zp)
    def _(r):
        for j in range(D // 16):
            zbuf[r, pl.ds(16 * j, 16)] = zero16

    for k in range(5):
        pltpu.sync_copy(zbuf, acc_sh.at[pl.ds(t * RPT + k * zp, zp)])

    plsc.subcore_barrier()

    base = (c * NS + t) * EPT

    @pl.loop(0, NCHUNK)
    def _(i):
        off = base + i * CH
        pltpu.sync_copy(src_hbm.at[pl.ds(off, CH)], idx2d.at[0])
        pltpu.sync_copy(dst_hbm.at[pl.ds(off, CH)], idx2d.at[1])
        pltpu.sync_copy(w16_hbm.at[pl.ds(off, CH)], wchunk)
        pltpu.sync_copy(xt_hbm.at[idx2d.at[0]], rows)

        @pl.loop(0, CH)
        def _(r):
            n = wchunk[r, :]
            for j in range(D // 16):
                sl = pl.ds(16 * j, 16)
                rows[r, sl] = rows[r, sl] * n

        pltpu.sync_copy(rows, acc_sh.at[idx2d.at[1]], add=True)

    plsc.subcore_barrier()
    for k in range(5):
        pltpu.sync_copy(acc_sh.at[pl.ds(t * RPT + k * zp, zp)], zbuf)
        pltpu.sync_copy(zbuf, out_hbm.at[c, pl.ds(t * RPT + k * zp, zp)])


def _prep_body(dp0_ref, dp1_ref, x_ref, dinv_ref, xt_ref):
    deg = dp0_ref[...] + dp1_ref[...] + 2.0
    dinv = lax.rsqrt(deg)
    dinv_ref[...] = dinv
    xt_ref[...] = x_ref[...] * dinv


def _final_body(xt_ref, p0_ref, p1_ref, dinv_ref, h0_ref,
                wzrh_ref, bzrh_ref, lzw1_ref, lzw2_ref, lrw1_ref, lrw2_ref,
                lhw1_ref, lhw2_ref, lzb_ref, lrb_ref, lhb_ref,
                linw_ref, linb_ref, H_ref, y_ref):
    f32 = jnp.float32
    ax = dinv_ref[...] * (p0_ref[...] + p1_ref[...] + 2.0 * xt_ref[...])
    czrh = jnp.dot(ax, wzrh_ref[...], preferred_element_type=f32) + bzrh_ref[...]
    cz = czrh[:, :D]
    cr = czrh[:, D:2 * D]
    ch = czrh[:, 2 * D:]
    h0 = h0_ref[...]
    Z = jax.nn.sigmoid(jnp.dot(cz, lzw1_ref[...], preferred_element_type=f32)
                       + jnp.dot(h0, lzw2_ref[...], preferred_element_type=f32)
                       + lzb_ref[...])
    R = jax.nn.sigmoid(jnp.dot(cr, lrw1_ref[...], preferred_element_type=f32)
                       + jnp.dot(h0, lrw2_ref[...], preferred_element_type=f32)
                       + lrb_ref[...])
    Ht = jnp.tanh(jnp.dot(ch, lhw1_ref[...], preferred_element_type=f32)
                  + jnp.dot(h0 * R, lhw2_ref[...], preferred_element_type=f32)
                  + lhb_ref[...])
    H = Z * h0 + (1.0 - Z) * Ht
    H_ref[...] = H
    y_ref[...] = (jnp.dot(jnp.maximum(H, 0.0), linw_ref[...],
                          preferred_element_type=f32) + linb_ref[...])


def kernel(x, edge_index, edge_weight, h0, W_z, b_z, W_r, b_r, W_h, b_h,
           lz_w, lz_b, lr_w, lr_b, lh_w, lh_b, lin_w, lin_b):
    src = edge_index[0]
    dst = edge_index[1]

    BW = 8000
    w16 = pl.pallas_call(
        _wpad_body,
        grid=(E // BW,),
        in_specs=[pl.BlockSpec((BW, 1), lambda i: (i, 0))],
        out_specs=pl.BlockSpec((BW, 16), lambda i: (i, 0)),
        out_shape=jax.ShapeDtypeStruct((E, 16), jnp.float32),
    )(edge_weight.reshape(E, 1))

    deg_p = _deg_call(dst, w16)
    dp0 = deg_p[0, :N, 0:1]
    dp1 = deg_p[1, :N, 0:1]

    dinv, xt = pl.pallas_call(
        _prep_body,
        out_shape=[jax.ShapeDtypeStruct((N, 1), jnp.float32),
                   jax.ShapeDtypeStruct((N, D), jnp.float32)],
    )(dp0, dp1, x)

    partials = _prop_call(xt, src, dst, w16)
    p0 = partials[0, :N]
    p1 = partials[1, :N]

    wzrh = jnp.concatenate([W_z.T, W_r.T, W_h.T], axis=1)
    bzrh = jnp.concatenate([b_z, b_r, b_h]).reshape(1, 3 * D)
    lzw1 = lz_w[:, :D].T
    lzw2 = lz_w[:, D:].T
    lrw1 = lr_w[:, :D].T
    lrw2 = lr_w[:, D:].T
    lhw1 = lh_w[:, :D].T
    lhw2 = lh_w[:, D:].T
    linw = jnp.concatenate([lin_w, jnp.zeros((1, D), jnp.float32)], axis=0).T
    linb = jnp.concatenate([lin_b, jnp.zeros((1,), jnp.float32)]).reshape(1, 8)

    B = 1000
    row = lambda w: pl.BlockSpec((B, w), lambda i: (i, 0))
    cst = lambda s: pl.BlockSpec(s, lambda i: (0, 0))
    H, y8 = pl.pallas_call(
        _final_body,
        grid=(N // B,),
        in_specs=[row(D), row(D), row(D), pl.BlockSpec((B, 1), lambda i: (i, 0)),
                  row(D),
                  cst((D, 3 * D)), cst((1, 3 * D)),
                  cst((D, D)), cst((D, D)), cst((D, D)), cst((D, D)),
                  cst((D, D)), cst((D, D)),
                  cst((1, D)), cst((1, D)), cst((1, D)),
                  cst((D, 8)), cst((1, 8))],
        out_specs=[row(D), pl.BlockSpec((B, 8), lambda i: (i, 0))],
        out_shape=[jax.ShapeDtypeStruct((N, D), jnp.float32),
                   jax.ShapeDtypeStruct((N, 8), jnp.float32)],
    )(xt, p0, p1, dinv, h0,
      wzrh, bzrh, lzw1, lzw2, lrw1, lrw2, lhw1, lhw2,
      lz_b.reshape(1, D), lr_b.reshape(1, D), lh_b.reshape(1, D),
      linw, linb)

    return (H, y8[:, :7])
